# baseline (speedup 1.0000x reference)
import jax
import jax.numpy as jnp
from jax.experimental import pallas as pl
from jax.experimental.pallas import tpu as pltpu

_NCHUNKS = 8


def _dma_copy(x_ref, o_ref, sems):
    lanes = x_ref.shape[1] // _NCHUNKS
    for i in range(_NCHUNKS):
        pltpu.make_async_copy(
            x_ref.at[:, pl.ds(i * lanes, lanes)],
            o_ref.at[:, pl.ds(i * lanes, lanes)],
            sems.at[i],
        ).start()
    for i in range(_NCHUNKS):
        pltpu.make_async_copy(
            x_ref.at[:, pl.ds(i * lanes, lanes)],
            o_ref.at[:, pl.ds(i * lanes, lanes)],
            sems.at[i],
        ).wait()


def kernel(rays):
    n, d = rays.shape
    t = rays.T
    out = pl.pallas_call(
        _dma_copy,
        in_specs=[pl.BlockSpec(memory_space=pl.ANY)],
        out_specs=pl.BlockSpec(memory_space=pl.ANY),
        out_shape=jax.ShapeDtypeStruct(t.shape, t.dtype),
        scratch_shapes=[pltpu.SemaphoreType.DMA((_NCHUNKS,))],
    )(t)
    return out.T


# speedup vs baseline: 49.0872x; 49.0872x over previous
import jax
import jax.numpy as jnp
from jax.experimental import pallas as pl


def _copy_block(x_ref, o_ref):
    o_ref[...] = x_ref[...]


def kernel(rays):
    n, d = rays.shape
    t = rays.T
    block_l = 262144
    grid = n // block_l
    out = pl.pallas_call(
        _copy_block,
        grid=(grid,),
        in_specs=[pl.BlockSpec((d, block_l), lambda i: (0, i))],
        out_specs=pl.BlockSpec((d, block_l), lambda i: (0, i)),
        out_shape=jax.ShapeDtypeStruct(t.shape, t.dtype),
    )(t)
    return out.T
